# baseline (speedup 1.0000x reference)
import jax
import jax.numpy as jnp
from jax import lax
from jax.experimental import pallas as pl
from jax.experimental.pallas import tpu as pltpu
from jax.experimental.pallas import tpu_sc as plsc

NUM_CORES = 2
NUM_SUBCORES = 16
LANES = 16
NUM_WORKERS = NUM_CORES * NUM_SUBCORES

BATCH = 16384
LATENT = 32
NUM_ROWS = 1000000
WIN = 128
NWIN = (NUM_ROWS + WIN - 1) // WIN
WORKERS_PER_TABLE = NUM_WORKERS // 2
WPW = (NWIN + WORKERS_PER_TABLE - 1) // WORKERS_PER_TABLE
NBINS = (WPW + 1) * LANES
HISTN = 8192
EXT_ROWS = BATCH + WIN
B_PER_W = BATCH // NUM_WORKERS
VECS = BATCH // LANES


def _stage_body(uidx_hbm, iidx_hbm, utabT_hbm, itabT_hbm, u_ext_hbm, i_ext_hbm,
                idxbuf, hist, begin, bucket, wb0, wb1, ostage, opos,
                sem_w0, sem_w1):
    wid = lax.axis_index("s") * NUM_CORES + lax.axis_index("c")
    tslot = lax.rem(wid, WORKERS_PER_TABLE)
    w0 = tslot * WPW
    nw = jnp.minimum(WPW, NWIN - w0)

    iota = lax.iota(jnp.int32, LANES)
    iota_hi = iota + LANES
    ones = jnp.ones((LANES,), jnp.int32)
    zeros = jnp.zeros((LANES,), jnp.int32)
    wbs = [wb0, wb1]
    sems = [sem_w0, sem_w1]

    def work(idx_hbm, tabT_hbm, ext_hbm):
        pltpu.sync_copy(idx_hbm, idxbuf.at[pl.ds(0, BATCH)])

        def zero(k, carry):
            hist[pl.ds(k * LANES, LANES)] = zeros
            return carry

        lax.fori_loop(0, HISTN // LANES, zero, 0)

        def hist_pass(k, carry):
            v = idxbuf[pl.ds(k * LANES, LANES)]
            loc = (v >> 7) - w0
            m = (loc >= 0) & (loc < nw)
            fb = jnp.where(m, loc * LANES + iota, 0)
            plsc.addupdate_scatter(hist, [fb], ones, mask=m)
            return carry

        lax.fori_loop(0, VECS, hist_pass, 0)

        def prefix(r, carry):
            h = hist[pl.ds(r * LANES, LANES)]
            cs = plsc.cumsum(h)
            ex = cs - h + carry
            begin[pl.ds(r * LANES, LANES)] = ex
            hist[pl.ds(r * LANES, LANES)] = ex
            return carry + cs[15]

        lax.fori_loop(0, NBINS // LANES, prefix, 0)

        def place(k, carry):
            v = idxbuf[pl.ds(k * LANES, LANES)]
            loc = (v >> 7) - w0
            m = (loc >= 0) & (loc < nw)
            fb = jnp.where(m, loc * LANES + iota, 0)
            ofs = plsc.load_gather(hist, [fb], mask=m)
            ofs = jnp.where(m, ofs, 0)
            packed = (k * LANES + iota) * WIN + (v & (WIN - 1))
            plsc.store_scatter(bucket, [ofs], packed, mask=m)
            plsc.addupdate_scatter(hist, [fb], ones, mask=m)
            return carry

        lax.fori_loop(0, VECS, place, 0)

        def opos_reset():
            orow = opos.at[0]
            for t in range(WIN // LANES):
                orow[pl.ds(t * LANES, LANES)] = BATCH + t * LANES + iota

        opos_reset()

        def fire(l_dyn, parity):
            off = pl.multiple_of((w0 + l_dyn) * WIN, 128)
            pltpu.async_copy(tabT_hbm.at[:, pl.ds(off, WIN)], wbs[parity],
                             sems[parity])

        def drain(parity):
            pltpu.make_async_copy(tabT_hbm.at[:, pl.ds(0, WIN)], wbs[parity],
                                  sems[parity]).wait()

        def flush():
            pltpu.sync_copy(ostage, ext_hbm.at[opos.at[0]])
            opos_reset()

        fire(0, 0)

        def window(l, slot):
            parity = lax.rem(l, 2)

            @pl.when(l + 1 < nw)
            def _():
                lax.cond(parity == 0, lambda: fire(l + 1, 1),
                         lambda: fire(l + 1, 0))

            lax.cond(parity == 0, lambda: drain(0), lambda: drain(1))

            bs = begin[pl.ds(l * LANES, LANES)][0]
            es = hist[pl.ds(l * LANES, LANES)][15]
            cnt = es - bs

            def item(i, slot):
                pks = bucket[pl.ds(bs + i, LANES)][0]
                pos = pks >> 7
                lane = pks & (WIN - 1)
                cvec = jnp.full((LANES,), lane, jnp.int32)

                def extract(par):
                    lo = plsc.load_gather(wbs[par], [iota, cvec])
                    hi = plsc.load_gather(wbs[par], [iota_hi, cvec])
                    ostage[slot, pl.ds(0, LANES)] = lo
                    ostage[slot, pl.ds(LANES, LANES)] = hi

                lax.cond(parity == 0, lambda: extract(0), lambda: extract(1))
                plsc.store_scatter(opos.at[0], [slot + iota],
                                   jnp.full((LANES,), pos, jnp.int32),
                                   mask=(iota == 0))

                @pl.when(slot == WIN - 1)
                def _():
                    flush()

                return lax.select(slot == WIN - 1, 0, slot + 1)

            return lax.fori_loop(0, cnt, item, slot)

        slot = lax.fori_loop(0, nw, window, 0)
        flush()
        del slot

    lax.cond(wid < WORKERS_PER_TABLE,
             lambda: work(uidx_hbm, utabT_hbm, u_ext_hbm),
             lambda: work(iidx_hbm, itabT_hbm, i_ext_hbm))


def _combine_body(u_ext_hbm, i_ext_hbm, w_hbm, b_hbm, out_hbm,
                  ubuf, ibuf, colbuf, out_v, wv, bv):
    wid = lax.axis_index("s") * NUM_CORES + lax.axis_index("c")
    base = wid * B_PER_W

    pltpu.sync_copy(w_hbm, wv)
    pltpu.sync_copy(b_hbm, bv)

    iota = lax.iota(jnp.int32, LANES)
    iota16 = iota * LANES
    w_lo = wv[pl.ds(0, LANES)]
    w_hi = wv[pl.ds(LANES, LANES)]
    b_vec = bv[...]

    for c in range(B_PER_W // WIN):
        pltpu.sync_copy(u_ext_hbm.at[pl.ds(base + c * WIN, WIN)], ubuf)
        pltpu.sync_copy(i_ext_hbm.at[pl.ds(base + c * WIN, WIN)], ibuf)

        def group(g, carry, c=c):
            j0 = g * LANES
            for e in range(LANES):
                j = j0 + e
                u_lo = ubuf[j, pl.ds(0, LANES)]
                u_hi = ubuf[j, pl.ds(LANES, LANES)]
                i_lo = ibuf[j, pl.ds(0, LANES)]
                i_hi = ibuf[j, pl.ds(LANES, LANES)]
                p = u_lo * i_lo * w_lo + u_hi * i_hi * w_hi
                plsc.store_scatter(colbuf, [iota16 + e], p)
            acc = colbuf[pl.ds(0, LANES)]
            for r in range(1, LANES):
                acc = acc + colbuf[pl.ds(r * LANES, LANES)]
            t = acc + b_vec
            sig = 1.0 / (1.0 + jnp.exp(-t))
            plsc.store_scatter(out_v, [c * WIN + g * LANES + iota], sig)
            return carry

        lax.fori_loop(0, WIN // LANES, group, 0)

    pltpu.sync_copy(out_v, out_hbm.at[pl.ds(base, B_PER_W)])


@jax.jit
def _gmf(user_indices, item_indices, utabT, itabT, w_flat, b_vec):
    mesh = plsc.VectorSubcoreMesh(core_axis_name="c", subcore_axis_name="s",
                                  num_cores=NUM_CORES, num_subcores=NUM_SUBCORES)
    stage = pl.kernel(
        _stage_body,
        out_type=[jax.ShapeDtypeStruct((EXT_ROWS, WIN), jnp.float32),
                  jax.ShapeDtypeStruct((EXT_ROWS, WIN), jnp.float32)],
        mesh=mesh,
        compiler_params=pltpu.CompilerParams(needs_layout_passes=False),
        scratch_types=[
            pltpu.VMEM((BATCH + LANES,), jnp.int32),
            pltpu.VMEM((HISTN,), jnp.int32),
            pltpu.VMEM((HISTN,), jnp.int32),
            pltpu.VMEM((BATCH + LANES,), jnp.int32),
            pltpu.VMEM((LATENT, WIN), jnp.float32),
            pltpu.VMEM((LATENT, WIN), jnp.float32),
            pltpu.VMEM((WIN, WIN), jnp.float32),
            pltpu.VMEM((1, WIN), jnp.int32),
            pltpu.SemaphoreType.DMA,
            pltpu.SemaphoreType.DMA,
        ],
    )
    u_ext, i_ext = stage(user_indices, item_indices, utabT, itabT)
    combine = pl.kernel(
        _combine_body,
        out_type=jax.ShapeDtypeStruct((BATCH,), jnp.float32),
        mesh=mesh,
        compiler_params=pltpu.CompilerParams(needs_layout_passes=False),
        scratch_types=[
            pltpu.VMEM((WIN, WIN), jnp.float32),
            pltpu.VMEM((WIN, WIN), jnp.float32),
            pltpu.VMEM((LANES * LANES,), jnp.float32),
            pltpu.VMEM((B_PER_W,), jnp.float32),
            pltpu.VMEM((LATENT,), jnp.float32),
            pltpu.VMEM((LANES,), jnp.float32),
        ],
    )
    return combine(u_ext, i_ext, w_flat, b_vec)


def kernel(user_indices, item_indices, user_table, item_table, W, b):
    utabT = user_table.T
    itabT = item_table.T
    w_flat = W.reshape(LATENT)
    b_vec = jnp.broadcast_to(b, (LANES,))
    out = _gmf(user_indices.astype(jnp.int32), item_indices.astype(jnp.int32),
               utabT, itabT, w_flat, b_vec)
    return out.reshape(BATCH, 1)


# speedup vs baseline: 1.4877x; 1.4877x over previous
import jax
import jax.numpy as jnp
from jax import lax
from jax.experimental import pallas as pl
from jax.experimental.pallas import tpu as pltpu
from jax.experimental.pallas import tpu_sc as plsc

NUM_CORES = 2
NUM_SUBCORES = 16
LANES = 16
NUM_WORKERS = NUM_CORES * NUM_SUBCORES

BATCH = 16384
LATENT = 32
B_PER_W = BATCH // NUM_WORKERS
GROUPS = B_PER_W // LANES
WIN = 128
QE = 4
QUARTERS = B_PER_W // QE
QBUF = QE * WIN


def _gmf_body(uidx_hbm, iidx_hbm, utabT_hbm, itabT_hbm, w_hbm, b_hbm, out_hbm,
              iu_raw, ii_raw, ub0, ub1, ub2, ib0, ib1, ib2, colbuf, out_v, wv, bv,
              su0, su1, su2, si0, si1, si2):
    wid = lax.axis_index("s") * NUM_CORES + lax.axis_index("c")
    base = wid * B_PER_W

    pltpu.sync_copy(uidx_hbm.at[pl.ds(base, B_PER_W)],
                    iu_raw.at[pl.ds(0, B_PER_W)])
    pltpu.sync_copy(iidx_hbm.at[pl.ds(base, B_PER_W)],
                    ii_raw.at[pl.ds(0, B_PER_W)])
    pltpu.sync_copy(w_hbm, wv)
    pltpu.sync_copy(b_hbm, bv)

    ubs, ibs = [ub0, ub1, ub2], [ib0, ib1, ib2]
    sus, sis = [su0, su1, su2], [si0, si1, si2]

    def fire(q_dyn, parity):
        vu = iu_raw[pl.ds(q_dyn * QE, LANES)]
        vi = ii_raw[pl.ds(q_dyn * QE, LANES)]
        for e in range(QE):
            offu = pl.multiple_of((vu[e] >> 7) * WIN, 128)
            offi = pl.multiple_of((vi[e] >> 7) * WIN, 128)
            pltpu.async_copy(utabT_hbm.at[:, pl.ds(offu, WIN)],
                             ubs[parity].at[:, pl.ds(e * WIN, WIN)],
                             sus[parity])
            pltpu.async_copy(itabT_hbm.at[:, pl.ds(offi, WIN)],
                             ibs[parity].at[:, pl.ds(e * WIN, WIN)],
                             sis[parity])

    def drain(parity):
        pltpu.make_async_copy(utabT_hbm.at[:, pl.ds(0, QBUF)],
                              ubs[parity], sus[parity]).wait()
        pltpu.make_async_copy(itabT_hbm.at[:, pl.ds(0, QBUF)],
                              ibs[parity], sis[parity]).wait()

    iota = lax.iota(jnp.int32, LANES)
    iota_hi = iota + LANES
    iota16 = iota * LANES
    w_lo = wv[pl.ds(0, LANES)]
    w_hi = wv[pl.ds(LANES, LANES)]
    b_vec = bv[...]

    fire(0, 0)
    fire(1, 1)

    def quarter(q, carry):
        parity = lax.rem(q, 3)

        @pl.when(q < QUARTERS - 2)
        def _():
            nxt = lax.rem(q + 2, 3)
            lax.switch(nxt, [lambda: fire(q + 2, 0), lambda: fire(q + 2, 1),
                             lambda: fire(q + 2, 2)])

        lax.switch(parity, [lambda: drain(0), lambda: drain(1),
                            lambda: drain(2)])

        vu = iu_raw[pl.ds(q * QE, LANES)]
        vi = ii_raw[pl.ds(q * QE, LANES)]
        e_base = lax.rem(q, 4) * QE

        def extract(par):
            ub, ib = ubs[par], ibs[par]
            for e in range(QE):
                cu = jnp.full((LANES,), e * WIN, jnp.int32) + (vu[e] & 127)
                ci = jnp.full((LANES,), e * WIN, jnp.int32) + (vi[e] & 127)
                u_lo = plsc.load_gather(ub, [iota, cu])
                u_hi = plsc.load_gather(ub, [iota_hi, cu])
                i_lo = plsc.load_gather(ib, [iota, ci])
                i_hi = plsc.load_gather(ib, [iota_hi, ci])
                p = u_lo * i_lo * w_lo + u_hi * i_hi * w_hi
                plsc.store_scatter(colbuf, [iota16 + (e_base + e)], p)

        lax.switch(parity, [lambda: extract(0), lambda: extract(1),
                            lambda: extract(2)])

        @pl.when(lax.rem(q, 4) == 3)
        def _():
            acc = colbuf[pl.ds(0, LANES)]
            for r in range(1, LANES):
                acc = acc + colbuf[pl.ds(r * LANES, LANES)]
            t = acc + b_vec
            sig = 1.0 / (1.0 + jnp.exp(-t))
            plsc.store_scatter(out_v, [(q // 4) * LANES + iota], sig)

        return carry

    lax.fori_loop(0, QUARTERS, quarter, 0)

    pltpu.sync_copy(out_v, out_hbm.at[pl.ds(base, B_PER_W)])


@jax.jit
def _gmf(user_indices, item_indices, utabT, itabT, w_flat, b_vec):
    mesh = plsc.VectorSubcoreMesh(core_axis_name="c", subcore_axis_name="s",
                                  num_cores=NUM_CORES, num_subcores=NUM_SUBCORES)
    run = pl.kernel(
        _gmf_body,
        out_type=jax.ShapeDtypeStruct((BATCH,), jnp.float32),
        mesh=mesh,
        compiler_params=pltpu.CompilerParams(needs_layout_passes=False),
        scratch_types=[
            pltpu.VMEM((B_PER_W + LANES,), jnp.int32),
            pltpu.VMEM((B_PER_W + LANES,), jnp.int32),
            pltpu.VMEM((LATENT, QBUF), jnp.float32),
            pltpu.VMEM((LATENT, QBUF), jnp.float32),
            pltpu.VMEM((LATENT, QBUF), jnp.float32),
            pltpu.VMEM((LATENT, QBUF), jnp.float32),
            pltpu.VMEM((LATENT, QBUF), jnp.float32),
            pltpu.VMEM((LATENT, QBUF), jnp.float32),
            pltpu.VMEM((LANES * LANES,), jnp.float32),
            pltpu.VMEM((B_PER_W,), jnp.float32),
            pltpu.VMEM((LATENT,), jnp.float32),
            pltpu.VMEM((LANES,), jnp.float32),
            pltpu.SemaphoreType.DMA,
            pltpu.SemaphoreType.DMA,
            pltpu.SemaphoreType.DMA,
            pltpu.SemaphoreType.DMA,
            pltpu.SemaphoreType.DMA,
            pltpu.SemaphoreType.DMA,
        ],
    )
    return run(user_indices, item_indices, utabT, itabT, w_flat, b_vec)


def kernel(user_indices, item_indices, user_table, item_table, W, b):
    utabT = user_table.T
    itabT = item_table.T
    w_flat = W.reshape(LATENT)
    b_vec = jnp.broadcast_to(b, (LANES,))
    out = _gmf(user_indices.astype(jnp.int32), item_indices.astype(jnp.int32),
               utabT, itabT, w_flat, b_vec)
    return out.reshape(BATCH, 1)


# speedup vs baseline: 1.5533x; 1.0441x over previous
import jax
import jax.numpy as jnp
from jax import lax
from jax.experimental import pallas as pl
from jax.experimental.pallas import tpu as pltpu
from jax.experimental.pallas import tpu_sc as plsc

NUM_CORES = 2
NUM_SUBCORES = 16
LANES = 16
NUM_WORKERS = NUM_CORES * NUM_SUBCORES

BATCH = 16384
LATENT = 32
NUM_ROWS = 1000000
WIN = 128
NWIN = (NUM_ROWS + WIN - 1) // WIN
WORKERS_PER_TABLE = NUM_WORKERS // 2
WPW = (NWIN + WORKERS_PER_TABLE - 1) // WORKERS_PER_TABLE
NBINS = (WPW + 1) * LANES
HISTN = 8192
EXT_ROWS = BATCH + WIN
B_PER_W = BATCH // NUM_WORKERS
VECS = BATCH // LANES
NBUF = 5


def _stage_body(uidx_hbm, iidx_hbm, utabT_hbm, itabT_hbm, u_ext_hbm, i_ext_hbm,
                idxbuf, hist, begin, bucket, wb0, wb1, wb2, wb3, wb4,
                ostage, opos, sem_w0, sem_w1, sem_w2, sem_w3, sem_w4):
    wid = lax.axis_index("s") * NUM_CORES + lax.axis_index("c")
    tslot = lax.rem(wid, WORKERS_PER_TABLE)
    w0 = tslot * WPW
    nw = jnp.minimum(WPW, NWIN - w0)

    iota = lax.iota(jnp.int32, LANES)
    iota_hi = iota + LANES
    ones = jnp.ones((LANES,), jnp.int32)
    zeros = jnp.zeros((LANES,), jnp.int32)
    wbs = [wb0, wb1, wb2, wb3, wb4]
    sems = [sem_w0, sem_w1, sem_w2, sem_w3, sem_w4]

    def work(idx_hbm, tabT_hbm, ext_hbm):
        pltpu.sync_copy(idx_hbm, idxbuf.at[pl.ds(0, BATCH)])

        def zero(k, carry):
            hist[pl.ds(k * LANES, LANES)] = zeros
            return carry

        lax.fori_loop(0, HISTN // LANES, zero, 0)

        def hist_pass(k, carry):
            v = idxbuf[pl.ds(k * LANES, LANES)]
            loc = (v >> 7) - w0
            m = (loc >= 0) & (loc < nw)
            fb = jnp.where(m, loc * LANES + iota, 0)
            plsc.addupdate_scatter(hist, [fb], ones, mask=m)
            return carry

        lax.fori_loop(0, VECS, hist_pass, 0)

        def prefix(r, carry):
            h = hist[pl.ds(r * LANES, LANES)]
            cs = plsc.cumsum(h)
            ex = cs - h + carry
            begin[pl.ds(r * LANES, LANES)] = ex
            hist[pl.ds(r * LANES, LANES)] = ex
            return carry + cs[15]

        lax.fori_loop(0, NBINS // LANES, prefix, 0)

        def place(k, carry):
            v = idxbuf[pl.ds(k * LANES, LANES)]
            loc = (v >> 7) - w0
            m = (loc >= 0) & (loc < nw)
            fb = jnp.where(m, loc * LANES + iota, 0)
            ofs = plsc.load_gather(hist, [fb], mask=m)
            ofs = jnp.where(m, ofs, 0)
            packed = (k * LANES + iota) * WIN + (v & (WIN - 1))
            plsc.store_scatter(bucket, [ofs], packed, mask=m)
            plsc.addupdate_scatter(hist, [fb], ones, mask=m)
            return carry

        lax.fori_loop(0, VECS, place, 0)

        def opos_reset():
            orow = opos.at[0]
            for t in range(WIN // LANES):
                orow[pl.ds(t * LANES, LANES)] = BATCH + t * LANES + iota

        opos_reset()

        def fire(l_dyn, parity):
            off = pl.multiple_of((w0 + l_dyn) * WIN, 128)
            pltpu.async_copy(tabT_hbm.at[:, pl.ds(off, WIN)], wbs[parity],
                             sems[parity])

        def drain(parity):
            pltpu.make_async_copy(tabT_hbm.at[:, pl.ds(0, WIN)], wbs[parity],
                                  sems[parity]).wait()

        def flush():
            pltpu.sync_copy(ostage, ext_hbm.at[opos.at[0]])
            opos_reset()

        for p in range(NBUF - 1):
            fire(p, p)

        def window(l, slot):
            parity = lax.rem(l, NBUF)

            @pl.when(l + NBUF - 1 < nw)
            def _():
                nxt = lax.rem(l + NBUF - 1, NBUF)
                lax.switch(nxt, [lambda i=i: fire(l + NBUF - 1, i)
                                 for i in range(NBUF)])

            lax.switch(parity, [lambda i=i: drain(i) for i in range(NBUF)])

            bs = begin[pl.ds(l * LANES, LANES)][0]
            es = hist[pl.ds(l * LANES, LANES)][15]
            cnt = es - bs

            def item(i, slot):
                pks = bucket[pl.ds(bs + i, LANES)][0]
                pos = pks >> 7
                lane = pks & (WIN - 1)
                cvec = jnp.full((LANES,), lane, jnp.int32)

                def extract(par):
                    lo = plsc.load_gather(wbs[par], [iota, cvec])
                    hi = plsc.load_gather(wbs[par], [iota_hi, cvec])
                    ostage[slot, pl.ds(0, LANES)] = lo
                    ostage[slot, pl.ds(LANES, LANES)] = hi

                lax.switch(parity, [lambda i=i: extract(i)
                                    for i in range(NBUF)])
                plsc.store_scatter(opos.at[0], [slot + iota],
                                   jnp.full((LANES,), pos, jnp.int32),
                                   mask=(iota == 0))

                @pl.when(slot == WIN - 1)
                def _():
                    flush()

                return lax.select(slot == WIN - 1, 0, slot + 1)

            return lax.fori_loop(0, cnt, item, slot)

        slot = lax.fori_loop(0, nw, window, 0)
        flush()
        del slot

    lax.cond(wid < WORKERS_PER_TABLE,
             lambda: work(uidx_hbm, utabT_hbm, u_ext_hbm),
             lambda: work(iidx_hbm, itabT_hbm, i_ext_hbm))


def _combine_body(u_ext_hbm, i_ext_hbm, w_hbm, b_hbm, out_hbm,
                  ubuf, ibuf, colbuf, out_v, wv, bv):
    wid = lax.axis_index("s") * NUM_CORES + lax.axis_index("c")
    base = wid * B_PER_W

    pltpu.sync_copy(w_hbm, wv)
    pltpu.sync_copy(b_hbm, bv)

    iota = lax.iota(jnp.int32, LANES)
    iota16 = iota * LANES
    w_lo = wv[pl.ds(0, LANES)]
    w_hi = wv[pl.ds(LANES, LANES)]
    b_vec = bv[...]

    for c in range(B_PER_W // WIN):
        pltpu.sync_copy(u_ext_hbm.at[pl.ds(base + c * WIN, WIN)], ubuf)
        pltpu.sync_copy(i_ext_hbm.at[pl.ds(base + c * WIN, WIN)], ibuf)

        def group(g, carry, c=c):
            j0 = g * LANES
            for e in range(LANES):
                j = j0 + e
                u_lo = ubuf[j, pl.ds(0, LANES)]
                u_hi = ubuf[j, pl.ds(LANES, LANES)]
                i_lo = ibuf[j, pl.ds(0, LANES)]
                i_hi = ibuf[j, pl.ds(LANES, LANES)]
                p = u_lo * i_lo * w_lo + u_hi * i_hi * w_hi
                plsc.store_scatter(colbuf, [iota16 + e], p)
            acc = colbuf[pl.ds(0, LANES)]
            for r in range(1, LANES):
                acc = acc + colbuf[pl.ds(r * LANES, LANES)]
            t = acc + b_vec
            sig = 1.0 / (1.0 + jnp.exp(-t))
            plsc.store_scatter(out_v, [c * WIN + g * LANES + iota], sig)
            return carry

        lax.fori_loop(0, WIN // LANES, group, 0)

    pltpu.sync_copy(out_v, out_hbm.at[pl.ds(base, B_PER_W)])


@jax.jit
def _gmf(user_indices, item_indices, utabT, itabT, w_flat, b_vec):
    mesh = plsc.VectorSubcoreMesh(core_axis_name="c", subcore_axis_name="s",
                                  num_cores=NUM_CORES, num_subcores=NUM_SUBCORES)
    stage = pl.kernel(
        _stage_body,
        out_type=[jax.ShapeDtypeStruct((EXT_ROWS, WIN), jnp.float32),
                  jax.ShapeDtypeStruct((EXT_ROWS, WIN), jnp.float32)],
        mesh=mesh,
        compiler_params=pltpu.CompilerParams(needs_layout_passes=False),
        scratch_types=[
            pltpu.VMEM((BATCH + LANES,), jnp.int32),
            pltpu.VMEM((HISTN,), jnp.int32),
            pltpu.VMEM((HISTN,), jnp.int32),
            pltpu.VMEM((BATCH + LANES,), jnp.int32),
            pltpu.VMEM((LATENT, WIN), jnp.float32),
            pltpu.VMEM((LATENT, WIN), jnp.float32),
            pltpu.VMEM((LATENT, WIN), jnp.float32),
            pltpu.VMEM((LATENT, WIN), jnp.float32),
            pltpu.VMEM((LATENT, WIN), jnp.float32),
            pltpu.VMEM((WIN, WIN), jnp.float32),
            pltpu.VMEM((1, WIN), jnp.int32),
            pltpu.SemaphoreType.DMA,
            pltpu.SemaphoreType.DMA,
            pltpu.SemaphoreType.DMA,
            pltpu.SemaphoreType.DMA,
            pltpu.SemaphoreType.DMA,
        ],
    )
    u_ext, i_ext = stage(user_indices, item_indices, utabT, itabT)
    combine = pl.kernel(
        _combine_body,
        out_type=jax.ShapeDtypeStruct((BATCH,), jnp.float32),
        mesh=mesh,
        compiler_params=pltpu.CompilerParams(needs_layout_passes=False),
        scratch_types=[
            pltpu.VMEM((WIN, WIN), jnp.float32),
            pltpu.VMEM((WIN, WIN), jnp.float32),
            pltpu.VMEM((LANES * LANES,), jnp.float32),
            pltpu.VMEM((B_PER_W,), jnp.float32),
            pltpu.VMEM((LATENT,), jnp.float32),
            pltpu.VMEM((LANES,), jnp.float32),
        ],
    )
    return combine(u_ext, i_ext, w_flat, b_vec)


def kernel(user_indices, item_indices, user_table, item_table, W, b):
    utabT = user_table.T
    itabT = item_table.T
    w_flat = W.reshape(LATENT)
    b_vec = jnp.broadcast_to(b, (LANES,))
    out = _gmf(user_indices.astype(jnp.int32), item_indices.astype(jnp.int32),
               utabT, itabT, w_flat, b_vec)
    return out.reshape(BATCH, 1)


# speedup vs baseline: 1.6241x; 1.0456x over previous
import jax
import jax.numpy as jnp
from jax import lax
from jax.experimental import pallas as pl
from jax.experimental.pallas import tpu as pltpu
from jax.experimental.pallas import tpu_sc as plsc

NUM_CORES = 2
NUM_SUBCORES = 16
LANES = 16
NUM_WORKERS = NUM_CORES * NUM_SUBCORES

BATCH = 16384
LATENT = 32
NUM_ROWS = 1000000
WIN = 128
NWIN = (NUM_ROWS + WIN - 1) // WIN
WORKERS_PER_TABLE = NUM_WORKERS // 2
WPW = (NWIN + WORKERS_PER_TABLE - 1) // WORKERS_PER_TABLE
NBINS = (WPW + 1) * LANES
HISTN = 8192
EXT_ROWS = BATCH + WIN
B_PER_W = BATCH // NUM_WORKERS
VECS = BATCH // LANES
NBUF = 5


def _stage_body(uidx_hbm, iidx_hbm, utabT_hbm, itabT_hbm, u_ext_hbm, i_ext_hbm,
                idxbuf, hist, begin, bucket, wb0, wb1, wb2, wb3, wb4,
                ostage, opos, sem_w0, sem_w1, sem_w2, sem_w3, sem_w4):
    wid = lax.axis_index("s") * NUM_CORES + lax.axis_index("c")
    tslot = lax.rem(wid, WORKERS_PER_TABLE)
    w0 = tslot * WPW
    nw = jnp.minimum(WPW, NWIN - w0)

    iota = lax.iota(jnp.int32, LANES)
    iota_hi = iota + LANES
    ones = jnp.ones((LANES,), jnp.int32)
    zeros = jnp.zeros((LANES,), jnp.int32)
    wbs = [wb0, wb1, wb2, wb3, wb4]
    sems = [sem_w0, sem_w1, sem_w2, sem_w3, sem_w4]

    def work(idx_hbm, tabT_hbm, ext_hbm):
        pltpu.sync_copy(idx_hbm, idxbuf.at[pl.ds(0, BATCH)])

        def zero(k, carry):
            hist[pl.ds(k * LANES, LANES)] = zeros
            return carry

        lax.fori_loop(0, HISTN // LANES, zero, 0)

        def hist_pass(k, carry):
            v = idxbuf[pl.ds(k * LANES, LANES)]
            loc = (v >> 7) - w0
            m = (loc >= 0) & (loc < nw)
            fb = jnp.where(m, loc * LANES + iota, 0)
            plsc.addupdate_scatter(hist, [fb], ones, mask=m)
            return carry

        lax.fori_loop(0, VECS, hist_pass, 0)

        def prefix(r, carry):
            h = hist[pl.ds(r * LANES, LANES)]
            cs = plsc.cumsum(h)
            ex = cs - h + carry
            begin[pl.ds(r * LANES, LANES)] = ex
            hist[pl.ds(r * LANES, LANES)] = ex
            return carry + cs[15]

        lax.fori_loop(0, NBINS // LANES, prefix, 0)

        def place(k, carry):
            v = idxbuf[pl.ds(k * LANES, LANES)]
            loc = (v >> 7) - w0
            m = (loc >= 0) & (loc < nw)
            fb = jnp.where(m, loc * LANES + iota, 0)
            ofs = plsc.load_gather(hist, [fb], mask=m)
            ofs = jnp.where(m, ofs, 0)
            packed = (k * LANES + iota) * WIN + (v & (WIN - 1))
            plsc.store_scatter(bucket, [ofs], packed, mask=m)
            plsc.addupdate_scatter(hist, [fb], ones, mask=m)
            return carry

        lax.fori_loop(0, VECS, place, 0)

        def opos_reset():
            orow = opos.at[0]
            for t in range(WIN // LANES):
                orow[pl.ds(t * LANES, LANES)] = BATCH + t * LANES + iota

        opos_reset()

        def fire(l_dyn, parity):
            off = pl.multiple_of((w0 + l_dyn) * WIN, 128)
            pltpu.async_copy(tabT_hbm.at[:, pl.ds(off, WIN)], wbs[parity],
                             sems[parity])

        def drain(parity):
            pltpu.make_async_copy(tabT_hbm.at[:, pl.ds(0, WIN)], wbs[parity],
                                  sems[parity]).wait()

        def flush():
            pltpu.sync_copy(ostage, ext_hbm.at[opos.at[0]])
            opos_reset()

        def wcount(l_dyn):
            bs = begin[pl.ds(l_dyn * LANES, LANES)][0]
            es = hist[pl.ds(l_dyn * LANES, LANES)][15]
            return bs, es - bs

        for p in range(NBUF - 1):
            _, cnt_p = wcount(p)

            @pl.when((p < nw) & (cnt_p > 0))
            def _(p=p):
                fire(p, p)

        def window(l, slot):
            parity = lax.rem(l, NBUF)
            bs, cnt = wcount(l)

            @pl.when(l + NBUF - 1 < nw)
            def _():
                _, cnt_n = wcount(l + NBUF - 1)

                @pl.when(cnt_n > 0)
                def _():
                    nxt = lax.rem(l + NBUF - 1, NBUF)
                    lax.switch(nxt, [lambda i=i: fire(l + NBUF - 1, i)
                                     for i in range(NBUF)])

            def process():
                def run(par):
                    drain(par)

                    def item(i, slot):
                        pks = bucket[pl.ds(bs + i, LANES)][0]
                        pos = pks >> 7
                        lane = pks & (WIN - 1)
                        cvec = jnp.full((LANES,), lane, jnp.int32)
                        lo = plsc.load_gather(wbs[par], [iota, cvec])
                        hi = plsc.load_gather(wbs[par], [iota_hi, cvec])
                        ostage[slot, pl.ds(0, LANES)] = lo
                        ostage[slot, pl.ds(LANES, LANES)] = hi
                        plsc.store_scatter(opos.at[0], [slot + iota],
                                           jnp.full((LANES,), pos, jnp.int32),
                                           mask=(iota == 0))

                        @pl.when(slot == WIN - 1)
                        def _():
                            flush()

                        return lax.select(slot == WIN - 1, 0, slot + 1)

                    return lax.fori_loop(0, cnt, item, slot)

                return lax.switch(parity, [lambda i=i: run(i)
                                           for i in range(NBUF)])

            return lax.cond(cnt > 0, process, lambda: slot)

        slot = lax.fori_loop(0, nw, window, 0)
        flush()
        del slot

    lax.cond(wid < WORKERS_PER_TABLE,
             lambda: work(uidx_hbm, utabT_hbm, u_ext_hbm),
             lambda: work(iidx_hbm, itabT_hbm, i_ext_hbm))


def _combine_body(u_ext_hbm, i_ext_hbm, w_hbm, b_hbm, out_hbm,
                  ubuf, ibuf, colbuf, out_v, wv, bv):
    wid = lax.axis_index("s") * NUM_CORES + lax.axis_index("c")
    base = wid * B_PER_W

    pltpu.sync_copy(w_hbm, wv)
    pltpu.sync_copy(b_hbm, bv)

    iota = lax.iota(jnp.int32, LANES)
    iota16 = iota * LANES
    w_lo = wv[pl.ds(0, LANES)]
    w_hi = wv[pl.ds(LANES, LANES)]
    b_vec = bv[...]

    for c in range(B_PER_W // WIN):
        pltpu.sync_copy(u_ext_hbm.at[pl.ds(base + c * WIN, WIN)], ubuf)
        pltpu.sync_copy(i_ext_hbm.at[pl.ds(base + c * WIN, WIN)], ibuf)

        def group(g, carry, c=c):
            j0 = g * LANES
            for e in range(LANES):
                j = j0 + e
                u_lo = ubuf[j, pl.ds(0, LANES)]
                u_hi = ubuf[j, pl.ds(LANES, LANES)]
                i_lo = ibuf[j, pl.ds(0, LANES)]
                i_hi = ibuf[j, pl.ds(LANES, LANES)]
                p = u_lo * i_lo * w_lo + u_hi * i_hi * w_hi
                plsc.store_scatter(colbuf, [iota16 + e], p)
            acc = colbuf[pl.ds(0, LANES)]
            for r in range(1, LANES):
                acc = acc + colbuf[pl.ds(r * LANES, LANES)]
            t = acc + b_vec
            sig = 1.0 / (1.0 + jnp.exp(-t))
            plsc.store_scatter(out_v, [c * WIN + g * LANES + iota], sig)
            return carry

        lax.fori_loop(0, WIN // LANES, group, 0)

    pltpu.sync_copy(out_v, out_hbm.at[pl.ds(base, B_PER_W)])


@jax.jit
def _gmf(user_indices, item_indices, utabT, itabT, w_flat, b_vec):
    mesh = plsc.VectorSubcoreMesh(core_axis_name="c", subcore_axis_name="s",
                                  num_cores=NUM_CORES, num_subcores=NUM_SUBCORES)
    stage = pl.kernel(
        _stage_body,
        out_type=[jax.ShapeDtypeStruct((EXT_ROWS, WIN), jnp.float32),
                  jax.ShapeDtypeStruct((EXT_ROWS, WIN), jnp.float32)],
        mesh=mesh,
        compiler_params=pltpu.CompilerParams(needs_layout_passes=False),
        scratch_types=[
            pltpu.VMEM((BATCH + LANES,), jnp.int32),
            pltpu.VMEM((HISTN,), jnp.int32),
            pltpu.VMEM((HISTN,), jnp.int32),
            pltpu.VMEM((BATCH + LANES,), jnp.int32),
            pltpu.VMEM((LATENT, WIN), jnp.float32),
            pltpu.VMEM((LATENT, WIN), jnp.float32),
            pltpu.VMEM((LATENT, WIN), jnp.float32),
            pltpu.VMEM((LATENT, WIN), jnp.float32),
            pltpu.VMEM((LATENT, WIN), jnp.float32),
            pltpu.VMEM((WIN, WIN), jnp.float32),
            pltpu.VMEM((1, WIN), jnp.int32),
            pltpu.SemaphoreType.DMA,
            pltpu.SemaphoreType.DMA,
            pltpu.SemaphoreType.DMA,
            pltpu.SemaphoreType.DMA,
            pltpu.SemaphoreType.DMA,
        ],
    )
    u_ext, i_ext = stage(user_indices, item_indices, utabT, itabT)
    combine = pl.kernel(
        _combine_body,
        out_type=jax.ShapeDtypeStruct((BATCH,), jnp.float32),
        mesh=mesh,
        compiler_params=pltpu.CompilerParams(needs_layout_passes=False),
        scratch_types=[
            pltpu.VMEM((WIN, WIN), jnp.float32),
            pltpu.VMEM((WIN, WIN), jnp.float32),
            pltpu.VMEM((LANES * LANES,), jnp.float32),
            pltpu.VMEM((B_PER_W,), jnp.float32),
            pltpu.VMEM((LATENT,), jnp.float32),
            pltpu.VMEM((LANES,), jnp.float32),
        ],
    )
    return combine(u_ext, i_ext, w_flat, b_vec)


def kernel(user_indices, item_indices, user_table, item_table, W, b):
    utabT = user_table.T
    itabT = item_table.T
    w_flat = W.reshape(LATENT)
    b_vec = jnp.broadcast_to(b, (LANES,))
    out = _gmf(user_indices.astype(jnp.int32), item_indices.astype(jnp.int32),
               utabT, itabT, w_flat, b_vec)
    return out.reshape(BATCH, 1)


# speedup vs baseline: 1.7028x; 1.0484x over previous
import jax
import jax.numpy as jnp
from jax import lax
from jax.experimental import pallas as pl
from jax.experimental.pallas import tpu as pltpu
from jax.experimental.pallas import tpu_sc as plsc

NUM_CORES = 2
NUM_SUBCORES = 16
LANES = 16
NUM_WORKERS = NUM_CORES * NUM_SUBCORES

BATCH = 16384
LATENT = 32
NUM_ROWS = 1000000
WIN = 128
NWIN = (NUM_ROWS + WIN - 1) // WIN
WORKERS_PER_TABLE = NUM_WORKERS // 2
WPW = (NWIN + WORKERS_PER_TABLE - 1) // WORKERS_PER_TABLE
NBINS = (WPW + 1) * LANES
HISTN = 8192
EXT_ROWS = BATCH + WIN
B_PER_W = BATCH // NUM_WORKERS
VECS = BATCH // LANES
NBUF = 5


def _stage_body(uidx_hbm, iidx_hbm, utabT_hbm, itabT_hbm, u_ext_hbm, i_ext_hbm,
                idxbuf, hist, begin, bucket, wb0, wb1, wb2, wb3, wb4,
                ostage, opos, sem_w0, sem_w1, sem_w2, sem_w3, sem_w4,
                sem_f0, sem_f1):
    wid = lax.axis_index("s") * NUM_CORES + lax.axis_index("c")
    tslot = lax.rem(wid, WORKERS_PER_TABLE)
    w0 = tslot * WPW
    nw = jnp.minimum(WPW, NWIN - w0)

    iota = lax.iota(jnp.int32, LANES)
    iota_hi = iota + LANES
    ones = jnp.ones((LANES,), jnp.int32)
    zeros = jnp.zeros((LANES,), jnp.int32)
    wbs = [wb0, wb1, wb2, wb3, wb4]
    sems = [sem_w0, sem_w1, sem_w2, sem_w3, sem_w4]
    sem_f = [sem_f0, sem_f1]

    def work(idx_hbm, tabT_hbm, ext_hbm):
        pltpu.sync_copy(idx_hbm, idxbuf.at[pl.ds(0, BATCH)])

        def zero(k, carry):
            hist[pl.ds(k * LANES, LANES)] = zeros
            return carry

        lax.fori_loop(0, HISTN // LANES, zero, 0)

        def hist_pass(k, carry):
            v = idxbuf[pl.ds(k * LANES, LANES)]
            loc = (v >> 7) - w0
            m = (loc >= 0) & (loc < nw)
            fb = jnp.where(m, loc * LANES + iota, 0)
            plsc.addupdate_scatter(hist, [fb], ones, mask=m)
            return carry

        lax.fori_loop(0, VECS, hist_pass, 0)

        def prefix(r, carry):
            h = hist[pl.ds(r * LANES, LANES)]
            cs = plsc.cumsum(h)
            ex = cs - h + carry
            begin[pl.ds(r * LANES, LANES)] = ex
            hist[pl.ds(r * LANES, LANES)] = ex
            return carry + cs[15]

        lax.fori_loop(0, NBINS // LANES, prefix, 0)

        def place(k, carry):
            v = idxbuf[pl.ds(k * LANES, LANES)]
            loc = (v >> 7) - w0
            m = (loc >= 0) & (loc < nw)
            fb = jnp.where(m, loc * LANES + iota, 0)
            ofs = plsc.load_gather(hist, [fb], mask=m)
            ofs = jnp.where(m, ofs, 0)
            packed = (k * LANES + iota) * WIN + (v & (WIN - 1))
            plsc.store_scatter(bucket, [ofs], packed, mask=m)
            plsc.addupdate_scatter(hist, [fb], ones, mask=m)
            return carry

        lax.fori_loop(0, VECS, place, 0)

        def opos_reset(p):
            orow = opos.at[p]
            for t in range(WIN // LANES):
                orow[pl.ds(t * LANES, LANES)] = BATCH + t * LANES + iota

        opos_reset(0)
        opos_reset(1)

        def fire(l_dyn, parity):
            off = pl.multiple_of((w0 + l_dyn) * WIN, 128)
            pltpu.async_copy(tabT_hbm.at[:, pl.ds(off, WIN)], wbs[parity],
                             sems[parity])

        def drain(parity):
            pltpu.make_async_copy(tabT_hbm.at[:, pl.ds(0, WIN)], wbs[parity],
                                  sems[parity]).wait()

        def flush(k):
            def branch(p):
                pltpu.async_copy(ostage.at[pl.ds(p * WIN, WIN)],
                                 ext_hbm.at[opos.at[p]], sem_f[p])

                @pl.when(k >= 1)
                def _():
                    pltpu.make_async_copy(
                        ostage.at[pl.ds((1 - p) * WIN, WIN)],
                        ext_hbm.at[opos.at[1 - p]], sem_f[1 - p]).wait()
                    opos_reset(1 - p)

            lax.switch(lax.rem(k, 2), [lambda: branch(0), lambda: branch(1)])

        def wcount(l_dyn):
            bs = begin[pl.ds(l_dyn * LANES, LANES)][0]
            es = hist[pl.ds(l_dyn * LANES, LANES)][15]
            return bs, es - bs

        for p in range(NBUF - 1):
            _, cnt_p = wcount(p)

            @pl.when((p < nw) & (cnt_p > 0))
            def _(p=p):
                fire(p, p)

        def window(l, carry):
            slot, k = carry
            parity = lax.rem(l, NBUF)
            bs, cnt = wcount(l)

            @pl.when(l + NBUF - 1 < nw)
            def _():
                _, cnt_n = wcount(l + NBUF - 1)

                @pl.when(cnt_n > 0)
                def _():
                    nxt = lax.rem(l + NBUF - 1, NBUF)
                    lax.switch(nxt, [lambda i=i: fire(l + NBUF - 1, i)
                                     for i in range(NBUF)])

            def process():
                def run(par):
                    drain(par)

                    def item(i, carry):
                        slot, k = carry
                        bp = lax.rem(k, 2)
                        row = bp * WIN + slot
                        pks = bucket[pl.ds(bs + i, LANES)][0]
                        pos = pks >> 7
                        lane = pks & (WIN - 1)
                        cvec = jnp.full((LANES,), lane, jnp.int32)
                        lo = plsc.load_gather(wbs[par], [iota, cvec])
                        hi = plsc.load_gather(wbs[par], [iota_hi, cvec])
                        ostage[row, pl.ds(0, LANES)] = lo
                        ostage[row, pl.ds(LANES, LANES)] = hi
                        plsc.store_scatter(
                            opos, [jnp.full((LANES,), bp, jnp.int32),
                                   slot + iota],
                            jnp.full((LANES,), pos, jnp.int32),
                            mask=(iota == 0))

                        @pl.when(slot == WIN - 1)
                        def _():
                            flush(k)

                        return (lax.select(slot == WIN - 1, 0, slot + 1),
                                lax.select(slot == WIN - 1, k + 1, k))

                    return lax.fori_loop(0, cnt, item, (slot, k))

                return lax.switch(parity, [lambda i=i: run(i)
                                           for i in range(NBUF)])

            return lax.cond(cnt > 0, process, lambda: (slot, k))

        slot, k = lax.fori_loop(0, nw, window, (0, 0))
        flush(k)

        def final_drain(p):
            pltpu.make_async_copy(ostage.at[pl.ds(p * WIN, WIN)],
                                  ext_hbm.at[opos.at[p]], sem_f[p]).wait()

        lax.switch(lax.rem(k, 2), [lambda: final_drain(0),
                                   lambda: final_drain(1)])
        del slot

    lax.cond(wid < WORKERS_PER_TABLE,
             lambda: work(uidx_hbm, utabT_hbm, u_ext_hbm),
             lambda: work(iidx_hbm, itabT_hbm, i_ext_hbm))


def _combine_body(u_ext_hbm, i_ext_hbm, w_hbm, b_hbm, out_hbm,
                  ubuf, ubuf1, ibuf, ibuf1, colbuf, out_v, wv, bv,
                  sem_u, sem_i):
    wid = lax.axis_index("s") * NUM_CORES + lax.axis_index("c")
    base = wid * B_PER_W

    pltpu.sync_copy(w_hbm, wv)
    pltpu.sync_copy(b_hbm, bv)

    iota = lax.iota(jnp.int32, LANES)
    iota16 = iota * LANES
    w_lo = wv[pl.ds(0, LANES)]
    w_hi = wv[pl.ds(LANES, LANES)]
    b_vec = bv[...]

    ubufs, ibufs = [ubuf, ubuf1], [ibuf, ibuf1]
    nchunks = B_PER_W // WIN

    def fire2(c, p):
        return [pltpu.async_copy(u_ext_hbm.at[pl.ds(base + c * WIN, WIN)],
                                 ubufs[p], sem_u),
                pltpu.async_copy(i_ext_hbm.at[pl.ds(base + c * WIN, WIN)],
                                 ibufs[p], sem_i)]

    cps = fire2(0, 0)
    for c in range(nchunks):
        if c + 1 < nchunks:
            nxt = fire2(c + 1, (c + 1) % 2)
        for cp in cps:
            cp.wait()
        ub, ib = ubufs[c % 2], ibufs[c % 2]

        def group(g, carry, c=c, ub=ub, ib=ib):
            j0 = g * LANES
            for e in range(LANES):
                j = j0 + e
                u_lo = ub[j, pl.ds(0, LANES)]
                u_hi = ub[j, pl.ds(LANES, LANES)]
                i_lo = ib[j, pl.ds(0, LANES)]
                i_hi = ib[j, pl.ds(LANES, LANES)]
                p = u_lo * i_lo * w_lo + u_hi * i_hi * w_hi
                plsc.store_scatter(colbuf, [iota16 + e], p)
            acc = colbuf[pl.ds(0, LANES)]
            for r in range(1, LANES):
                acc = acc + colbuf[pl.ds(r * LANES, LANES)]
            t = acc + b_vec
            sig = 1.0 / (1.0 + jnp.exp(-t))
            plsc.store_scatter(out_v, [c * WIN + g * LANES + iota], sig)
            return carry

        lax.fori_loop(0, WIN // LANES, group, 0)
        if c + 1 < nchunks:
            cps = nxt

    pltpu.sync_copy(out_v, out_hbm.at[pl.ds(base, B_PER_W)])


@jax.jit
def _gmf(user_indices, item_indices, utabT, itabT, w_flat, b_vec):
    mesh = plsc.VectorSubcoreMesh(core_axis_name="c", subcore_axis_name="s",
                                  num_cores=NUM_CORES, num_subcores=NUM_SUBCORES)
    stage = pl.kernel(
        _stage_body,
        out_type=[jax.ShapeDtypeStruct((EXT_ROWS, WIN), jnp.float32),
                  jax.ShapeDtypeStruct((EXT_ROWS, WIN), jnp.float32)],
        mesh=mesh,
        compiler_params=pltpu.CompilerParams(needs_layout_passes=False),
        scratch_types=[
            pltpu.VMEM((BATCH + LANES,), jnp.int32),
            pltpu.VMEM((HISTN,), jnp.int32),
            pltpu.VMEM((HISTN,), jnp.int32),
            pltpu.VMEM((BATCH + LANES,), jnp.int32),
            pltpu.VMEM((LATENT, WIN), jnp.float32),
            pltpu.VMEM((LATENT, WIN), jnp.float32),
            pltpu.VMEM((LATENT, WIN), jnp.float32),
            pltpu.VMEM((LATENT, WIN), jnp.float32),
            pltpu.VMEM((LATENT, WIN), jnp.float32),
            pltpu.VMEM((2 * WIN, WIN), jnp.float32),
            pltpu.VMEM((2, WIN), jnp.int32),
            pltpu.SemaphoreType.DMA,
            pltpu.SemaphoreType.DMA,
            pltpu.SemaphoreType.DMA,
            pltpu.SemaphoreType.DMA,
            pltpu.SemaphoreType.DMA,
            pltpu.SemaphoreType.DMA,
            pltpu.SemaphoreType.DMA,
        ],
    )
    u_ext, i_ext = stage(user_indices, item_indices, utabT, itabT)
    combine = pl.kernel(
        _combine_body,
        out_type=jax.ShapeDtypeStruct((BATCH,), jnp.float32),
        mesh=mesh,
        compiler_params=pltpu.CompilerParams(needs_layout_passes=False),
        scratch_types=[
            pltpu.VMEM((WIN, WIN), jnp.float32),
            pltpu.VMEM((WIN, WIN), jnp.float32),
            pltpu.VMEM((WIN, WIN), jnp.float32),
            pltpu.VMEM((WIN, WIN), jnp.float32),
            pltpu.VMEM((LANES * LANES,), jnp.float32),
            pltpu.VMEM((B_PER_W,), jnp.float32),
            pltpu.VMEM((LATENT,), jnp.float32),
            pltpu.VMEM((LANES,), jnp.float32),
            pltpu.SemaphoreType.DMA,
            pltpu.SemaphoreType.DMA,
        ],
    )
    return combine(u_ext, i_ext, w_flat, b_vec)


def kernel(user_indices, item_indices, user_table, item_table, W, b):
    utabT = user_table.T
    itabT = item_table.T
    w_flat = W.reshape(LATENT)
    b_vec = jnp.broadcast_to(b, (LANES,))
    out = _gmf(user_indices.astype(jnp.int32), item_indices.astype(jnp.int32),
               utabT, itabT, w_flat, b_vec)
    return out.reshape(BATCH, 1)


# speedup vs baseline: 1.9070x; 1.1199x over previous
import jax
import jax.numpy as jnp
from jax import lax
from jax.experimental import pallas as pl
from jax.experimental.pallas import tpu as pltpu
from jax.experimental.pallas import tpu_sc as plsc

NUM_CORES = 2
NUM_SUBCORES = 16
LANES = 16
NUM_WORKERS = NUM_CORES * NUM_SUBCORES

BATCH = 16384
LATENT = 32
NUM_ROWS = 1000000
WIN = 128
NWIN = (NUM_ROWS + WIN - 1) // WIN
WORKERS_PER_TABLE = NUM_WORKERS // 2
WPW = (NWIN + WORKERS_PER_TABLE - 1) // WORKERS_PER_TABLE
NBINS = (WPW + 1) * LANES
HISTN = 8192
EXT_ROWS = BATCH + WIN
B_PER_W = BATCH // NUM_WORKERS
VECS = BATCH // LANES
NBUF = 7


def _stage_body(uidx_hbm, iidx_hbm, utabT_hbm, itabT_hbm, u_ext_hbm, i_ext_hbm,
                idxbuf, hist, begin, bucket, wb0, wb1, wb2, wb3, wb4, wb5, wb6,
                ostage, opos, sem_w0, sem_w1, sem_w2, sem_w3, sem_w4, sem_w5,
                sem_w6, sem_f0, sem_f1):
    wid = lax.axis_index("s") * NUM_CORES + lax.axis_index("c")
    tslot = lax.rem(wid, WORKERS_PER_TABLE)
    w0 = tslot * WPW
    nw = jnp.minimum(WPW, NWIN - w0)

    iota = lax.iota(jnp.int32, LANES)
    iota_hi = iota + LANES
    ones = jnp.ones((LANES,), jnp.int32)
    zeros = jnp.zeros((LANES,), jnp.int32)
    wbs = [wb0, wb1, wb2, wb3, wb4, wb5, wb6]
    sems = [sem_w0, sem_w1, sem_w2, sem_w3, sem_w4, sem_w5, sem_w6]
    sem_f = [sem_f0, sem_f1]

    def work(idx_hbm, tabT_hbm, ext_hbm):
        pltpu.sync_copy(idx_hbm, idxbuf.at[pl.ds(0, BATCH)])

        def zero(k, carry):
            hist[pl.ds(k * LANES, LANES)] = zeros
            return carry

        lax.fori_loop(0, HISTN // LANES, zero, 0)

        def hist_pass(k, carry):
            v = idxbuf[pl.ds(k * LANES, LANES)]
            loc = (v >> 7) - w0
            m = (loc >= 0) & (loc < nw)
            fb = jnp.where(m, loc * LANES + iota, 0)
            plsc.addupdate_scatter(hist, [fb], ones, mask=m)
            return carry

        lax.fori_loop(0, VECS, hist_pass, 0)

        def prefix(r, carry):
            h = hist[pl.ds(r * LANES, LANES)]
            cs = plsc.cumsum(h)
            ex = cs - h + carry
            begin[pl.ds(r * LANES, LANES)] = ex
            hist[pl.ds(r * LANES, LANES)] = ex
            return carry + cs[15]

        lax.fori_loop(0, NBINS // LANES, prefix, 0)

        def place(k, carry):
            v = idxbuf[pl.ds(k * LANES, LANES)]
            loc = (v >> 7) - w0
            m = (loc >= 0) & (loc < nw)
            fb = jnp.where(m, loc * LANES + iota, 0)
            ofs = plsc.load_gather(hist, [fb], mask=m)
            ofs = jnp.where(m, ofs, 0)
            packed = (k * LANES + iota) * WIN + (v & (WIN - 1))
            plsc.store_scatter(bucket, [ofs], packed, mask=m)
            plsc.addupdate_scatter(hist, [fb], ones, mask=m)
            return carry

        lax.fori_loop(0, VECS, place, 0)

        def opos_reset(p):
            orow = opos.at[p]
            for t in range(WIN // LANES):
                orow[pl.ds(t * LANES, LANES)] = BATCH + t * LANES + iota

        opos_reset(0)
        opos_reset(1)

        def fire(l_dyn, parity):
            off = pl.multiple_of((w0 + l_dyn) * WIN, 128)
            pltpu.async_copy(tabT_hbm.at[:, pl.ds(off, WIN)], wbs[parity],
                             sems[parity])

        def drain(parity):
            pltpu.make_async_copy(tabT_hbm.at[:, pl.ds(0, WIN)], wbs[parity],
                                  sems[parity]).wait()

        def flush(k):
            def branch(p):
                pltpu.async_copy(ostage.at[pl.ds(p * WIN, WIN)],
                                 ext_hbm.at[opos.at[p]], sem_f[p])

                @pl.when(k >= 1)
                def _():
                    pltpu.make_async_copy(
                        ostage.at[pl.ds((1 - p) * WIN, WIN)],
                        ext_hbm.at[opos.at[1 - p]], sem_f[1 - p]).wait()
                    opos_reset(1 - p)

            lax.switch(lax.rem(k, 2), [lambda: branch(0), lambda: branch(1)])

        def wcount(l_dyn):
            bs = begin[pl.ds(l_dyn * LANES, LANES)][0]
            es = hist[pl.ds(l_dyn * LANES, LANES)][15]
            return bs, es - bs

        for p in range(NBUF - 1):
            _, cnt_p = wcount(p)

            @pl.when((p < nw) & (cnt_p > 0))
            def _(p=p):
                fire(p, p)

        def window(l, carry):
            slot, k = carry
            parity = lax.rem(l, NBUF)
            bs, cnt = wcount(l)

            @pl.when(l + NBUF - 1 < nw)
            def _():
                _, cnt_n = wcount(l + NBUF - 1)

                @pl.when(cnt_n > 0)
                def _():
                    nxt = lax.rem(l + NBUF - 1, NBUF)
                    lax.switch(nxt, [lambda i=i: fire(l + NBUF - 1, i)
                                     for i in range(NBUF)])

            def process():
                def run(par):
                    drain(par)

                    def item(i, carry):
                        slot, k = carry
                        bp = lax.rem(k, 2)
                        row = bp * WIN + slot
                        pks = bucket[pl.ds(bs + i, LANES)][0]
                        pos = pks >> 7
                        lane = pks & (WIN - 1)
                        cvec = jnp.full((LANES,), lane, jnp.int32)
                        lo = plsc.load_gather(wbs[par], [iota, cvec])
                        hi = plsc.load_gather(wbs[par], [iota_hi, cvec])
                        ostage[row, pl.ds(0, LANES)] = lo
                        ostage[row, pl.ds(LANES, LANES)] = hi
                        plsc.store_scatter(
                            opos, [jnp.full((LANES,), bp, jnp.int32),
                                   slot + iota],
                            jnp.full((LANES,), pos, jnp.int32),
                            mask=(iota == 0))

                        @pl.when(slot == WIN - 1)
                        def _():
                            flush(k)

                        return (lax.select(slot == WIN - 1, 0, slot + 1),
                                lax.select(slot == WIN - 1, k + 1, k))

                    return lax.fori_loop(0, cnt, item, (slot, k))

                return lax.switch(parity, [lambda i=i: run(i)
                                           for i in range(NBUF)])

            return lax.cond(cnt > 0, process, lambda: (slot, k))

        slot, k = lax.fori_loop(0, nw, window, (0, 0))
        flush(k)

        def final_drain(p):
            pltpu.make_async_copy(ostage.at[pl.ds(p * WIN, WIN)],
                                  ext_hbm.at[opos.at[p]], sem_f[p]).wait()

        lax.switch(lax.rem(k, 2), [lambda: final_drain(0),
                                   lambda: final_drain(1)])
        del slot

    lax.cond(wid < WORKERS_PER_TABLE,
             lambda: work(uidx_hbm, utabT_hbm, u_ext_hbm),
             lambda: work(iidx_hbm, itabT_hbm, i_ext_hbm))


def _combine_body(u_ext_hbm, i_ext_hbm, w_hbm, b_hbm, out_hbm,
                  ubuf, ubuf1, ibuf, ibuf1, colbuf, out_v, wv, bv,
                  sem_u, sem_i):
    wid = lax.axis_index("s") * NUM_CORES + lax.axis_index("c")
    base = wid * B_PER_W

    pltpu.sync_copy(w_hbm, wv)
    pltpu.sync_copy(b_hbm, bv)

    iota = lax.iota(jnp.int32, LANES)
    iota16 = iota * LANES
    w_lo = wv[pl.ds(0, LANES)]
    w_hi = wv[pl.ds(LANES, LANES)]
    b_vec = bv[...]

    ubufs, ibufs = [ubuf, ubuf1], [ibuf, ibuf1]
    nchunks = B_PER_W // WIN

    def fire2(c, p):
        return [pltpu.async_copy(u_ext_hbm.at[pl.ds(base + c * WIN, WIN)],
                                 ubufs[p], sem_u),
                pltpu.async_copy(i_ext_hbm.at[pl.ds(base + c * WIN, WIN)],
                                 ibufs[p], sem_i)]

    cps = fire2(0, 0)
    for c in range(nchunks):
        if c + 1 < nchunks:
            nxt = fire2(c + 1, (c + 1) % 2)
        for cp in cps:
            cp.wait()
        ub, ib = ubufs[c % 2], ibufs[c % 2]

        def group(g, carry, c=c, ub=ub, ib=ib):
            j0 = g * LANES
            for e in range(LANES):
                j = j0 + e
                u_lo = ub[j, pl.ds(0, LANES)]
                u_hi = ub[j, pl.ds(LANES, LANES)]
                i_lo = ib[j, pl.ds(0, LANES)]
                i_hi = ib[j, pl.ds(LANES, LANES)]
                p = u_lo * i_lo * w_lo + u_hi * i_hi * w_hi
                plsc.store_scatter(colbuf, [iota16 + e], p)
            acc = colbuf[pl.ds(0, LANES)]
            for r in range(1, LANES):
                acc = acc + colbuf[pl.ds(r * LANES, LANES)]
            t = acc + b_vec
            sig = 1.0 / (1.0 + jnp.exp(-t))
            plsc.store_scatter(out_v, [c * WIN + g * LANES + iota], sig)
            return carry

        lax.fori_loop(0, WIN // LANES, group, 0)
        if c + 1 < nchunks:
            cps = nxt

    pltpu.sync_copy(out_v, out_hbm.at[pl.ds(base, B_PER_W)])


@jax.jit
def _gmf(user_indices, item_indices, utabT, itabT, w_flat, b_vec):
    mesh = plsc.VectorSubcoreMesh(core_axis_name="c", subcore_axis_name="s",
                                  num_cores=NUM_CORES, num_subcores=NUM_SUBCORES)
    stage = pl.kernel(
        _stage_body,
        out_type=[jax.ShapeDtypeStruct((EXT_ROWS, WIN), jnp.float32),
                  jax.ShapeDtypeStruct((EXT_ROWS, WIN), jnp.float32)],
        mesh=mesh,
        compiler_params=pltpu.CompilerParams(needs_layout_passes=False),
        scratch_types=[
            pltpu.VMEM((BATCH + LANES,), jnp.int32),
            pltpu.VMEM((HISTN,), jnp.int32),
            pltpu.VMEM((HISTN,), jnp.int32),
            pltpu.VMEM((BATCH + LANES,), jnp.int32),
            pltpu.VMEM((LATENT, WIN), jnp.float32),
            pltpu.VMEM((LATENT, WIN), jnp.float32),
            pltpu.VMEM((LATENT, WIN), jnp.float32),
            pltpu.VMEM((LATENT, WIN), jnp.float32),
            pltpu.VMEM((LATENT, WIN), jnp.float32),
            pltpu.VMEM((LATENT, WIN), jnp.float32),
            pltpu.VMEM((LATENT, WIN), jnp.float32),
            pltpu.VMEM((2 * WIN, WIN), jnp.float32),
            pltpu.VMEM((2, WIN), jnp.int32),
            pltpu.SemaphoreType.DMA,
            pltpu.SemaphoreType.DMA,
            pltpu.SemaphoreType.DMA,
            pltpu.SemaphoreType.DMA,
            pltpu.SemaphoreType.DMA,
            pltpu.SemaphoreType.DMA,
            pltpu.SemaphoreType.DMA,
            pltpu.SemaphoreType.DMA,
            pltpu.SemaphoreType.DMA,
        ],
    )
    u_ext, i_ext = stage(user_indices, item_indices, utabT, itabT)
    combine = pl.kernel(
        _combine_body,
        out_type=jax.ShapeDtypeStruct((BATCH,), jnp.float32),
        mesh=mesh,
        compiler_params=pltpu.CompilerParams(needs_layout_passes=False),
        scratch_types=[
            pltpu.VMEM((WIN, WIN), jnp.float32),
            pltpu.VMEM((WIN, WIN), jnp.float32),
            pltpu.VMEM((WIN, WIN), jnp.float32),
            pltpu.VMEM((WIN, WIN), jnp.float32),
            pltpu.VMEM((LANES * LANES,), jnp.float32),
            pltpu.VMEM((B_PER_W,), jnp.float32),
            pltpu.VMEM((LATENT,), jnp.float32),
            pltpu.VMEM((LANES,), jnp.float32),
            pltpu.SemaphoreType.DMA,
            pltpu.SemaphoreType.DMA,
        ],
    )
    return combine(u_ext, i_ext, w_flat, b_vec)


def kernel(user_indices, item_indices, user_table, item_table, W, b):
    utabT = user_table.T
    itabT = item_table.T
    w_flat = W.reshape(LATENT)
    b_vec = jnp.broadcast_to(b, (LANES,))
    out = _gmf(user_indices.astype(jnp.int32), item_indices.astype(jnp.int32),
               utabT, itabT, w_flat, b_vec)
    return out.reshape(BATCH, 1)


# speedup vs baseline: 1.9916x; 1.0444x over previous
import jax
import jax.numpy as jnp
from jax import lax
from jax.experimental import pallas as pl
from jax.experimental.pallas import tpu as pltpu
from jax.experimental.pallas import tpu_sc as plsc

NUM_CORES = 2
NUM_SUBCORES = 16
LANES = 16
NUM_WORKERS = NUM_CORES * NUM_SUBCORES

BATCH = 16384
LATENT = 32
NUM_ROWS = 1000000
WIN = 128
NWIN = (NUM_ROWS + WIN - 1) // WIN
WORKERS_PER_TABLE = NUM_WORKERS // 2
WPW = (NWIN + WORKERS_PER_TABLE - 1) // WORKERS_PER_TABLE
NBINS = (WPW + 1) * LANES
HISTN = 8192
EXT_ROWS = BATCH + WIN
B_PER_W = BATCH // NUM_WORKERS
VECS = BATCH // LANES
NBUF = 10


def _stage_body(uidx_hbm, iidx_hbm, utabT_hbm, itabT_hbm, u_ext_hbm, i_ext_hbm,
                idxbuf, hist, begin, bucket, wb0, wb1, wb2, wb3, wb4, wb5, wb6,
                wb7, wb8, wb9, ostage, opos, sem_w0, sem_w1, sem_w2, sem_w3,
                sem_w4, sem_w5, sem_w6, sem_w7, sem_w8, sem_w9,
                sem_f0, sem_f1):
    wid = lax.axis_index("s") * NUM_CORES + lax.axis_index("c")
    tslot = lax.rem(wid, WORKERS_PER_TABLE)
    w0 = tslot * WPW
    nw = jnp.minimum(WPW, NWIN - w0)

    iota = lax.iota(jnp.int32, LANES)
    iota_hi = iota + LANES
    ones = jnp.ones((LANES,), jnp.int32)
    zeros = jnp.zeros((LANES,), jnp.int32)
    wbs = [wb0, wb1, wb2, wb3, wb4, wb5, wb6, wb7, wb8, wb9]
    sems = [sem_w0, sem_w1, sem_w2, sem_w3, sem_w4, sem_w5, sem_w6, sem_w7,
            sem_w8, sem_w9]
    sem_f = [sem_f0, sem_f1]

    def work(idx_hbm, tabT_hbm, ext_hbm):
        pltpu.sync_copy(idx_hbm, idxbuf.at[pl.ds(0, BATCH)])

        def zero(k, carry):
            hist[pl.ds(k * LANES, LANES)] = zeros
            return carry

        lax.fori_loop(0, HISTN // LANES, zero, 0)

        def hist_pass(k, carry):
            v = idxbuf[pl.ds(k * LANES, LANES)]
            loc = (v >> 7) - w0
            m = (loc >= 0) & (loc < nw)
            fb = jnp.where(m, loc * LANES + iota, 0)
            plsc.addupdate_scatter(hist, [fb], ones, mask=m)
            return carry

        lax.fori_loop(0, VECS, hist_pass, 0)

        def prefix(r, carry):
            h = hist[pl.ds(r * LANES, LANES)]
            cs = plsc.cumsum(h)
            ex = cs - h + carry
            begin[pl.ds(r * LANES, LANES)] = ex
            hist[pl.ds(r * LANES, LANES)] = ex
            return carry + cs[15]

        lax.fori_loop(0, NBINS // LANES, prefix, 0)

        def place(k, carry):
            v = idxbuf[pl.ds(k * LANES, LANES)]
            loc = (v >> 7) - w0
            m = (loc >= 0) & (loc < nw)
            fb = jnp.where(m, loc * LANES + iota, 0)
            ofs = plsc.load_gather(hist, [fb], mask=m)
            ofs = jnp.where(m, ofs, 0)
            packed = (k * LANES + iota) * WIN + (v & (WIN - 1))
            plsc.store_scatter(bucket, [ofs], packed, mask=m)
            plsc.addupdate_scatter(hist, [fb], ones, mask=m)
            return carry

        lax.fori_loop(0, VECS, place, 0)

        def opos_reset(p):
            orow = opos.at[p]
            for t in range(WIN // LANES):
                orow[pl.ds(t * LANES, LANES)] = BATCH + t * LANES + iota

        opos_reset(0)
        opos_reset(1)

        def fire(l_dyn, parity):
            off = pl.multiple_of((w0 + l_dyn) * WIN, 128)
            pltpu.async_copy(tabT_hbm.at[:, pl.ds(off, WIN)], wbs[parity],
                             sems[parity])

        def drain(parity):
            pltpu.make_async_copy(tabT_hbm.at[:, pl.ds(0, WIN)], wbs[parity],
                                  sems[parity]).wait()

        def flush(k):
            def branch(p):
                pltpu.async_copy(ostage.at[pl.ds(p * WIN, WIN)],
                                 ext_hbm.at[opos.at[p]], sem_f[p])

                @pl.when(k >= 1)
                def _():
                    pltpu.make_async_copy(
                        ostage.at[pl.ds((1 - p) * WIN, WIN)],
                        ext_hbm.at[opos.at[1 - p]], sem_f[1 - p]).wait()
                    opos_reset(1 - p)

            lax.switch(lax.rem(k, 2), [lambda: branch(0), lambda: branch(1)])

        def wcount(l_dyn):
            bs = begin[pl.ds(l_dyn * LANES, LANES)][0]
            es = hist[pl.ds(l_dyn * LANES, LANES)][15]
            return bs, es - bs

        for p in range(NBUF - 1):
            _, cnt_p = wcount(p)

            @pl.when((p < nw) & (cnt_p > 0))
            def _(p=p):
                fire(p, p)

        def window(l, carry):
            slot, k = carry
            parity = lax.rem(l, NBUF)
            bs, cnt = wcount(l)

            @pl.when(l + NBUF - 1 < nw)
            def _():
                _, cnt_n = wcount(l + NBUF - 1)

                @pl.when(cnt_n > 0)
                def _():
                    nxt = lax.rem(l + NBUF - 1, NBUF)
                    lax.switch(nxt, [lambda i=i: fire(l + NBUF - 1, i)
                                     for i in range(NBUF)])

            def process():
                def run(par):
                    drain(par)

                    def item(i, carry):
                        slot, k = carry
                        bp = lax.rem(k, 2)
                        row = bp * WIN + slot
                        pks = bucket[pl.ds(bs + i, LANES)][0]
                        pos = pks >> 7
                        lane = pks & (WIN - 1)
                        cvec = jnp.full((LANES,), lane, jnp.int32)
                        lo = plsc.load_gather(wbs[par], [iota, cvec])
                        hi = plsc.load_gather(wbs[par], [iota_hi, cvec])
                        ostage[row, pl.ds(0, LANES)] = lo
                        ostage[row, pl.ds(LANES, LANES)] = hi
                        plsc.store_scatter(
                            opos, [jnp.full((LANES,), bp, jnp.int32),
                                   slot + iota],
                            jnp.full((LANES,), pos, jnp.int32),
                            mask=(iota == 0))

                        @pl.when(slot == WIN - 1)
                        def _():
                            flush(k)

                        return (lax.select(slot == WIN - 1, 0, slot + 1),
                                lax.select(slot == WIN - 1, k + 1, k))

                    return lax.fori_loop(0, cnt, item, (slot, k))

                return lax.switch(parity, [lambda i=i: run(i)
                                           for i in range(NBUF)])

            return lax.cond(cnt > 0, process, lambda: (slot, k))

        slot, k = lax.fori_loop(0, nw, window, (0, 0))
        flush(k)

        def final_drain(p):
            pltpu.make_async_copy(ostage.at[pl.ds(p * WIN, WIN)],
                                  ext_hbm.at[opos.at[p]], sem_f[p]).wait()

        lax.switch(lax.rem(k, 2), [lambda: final_drain(0),
                                   lambda: final_drain(1)])
        del slot

    lax.cond(wid < WORKERS_PER_TABLE,
             lambda: work(uidx_hbm, utabT_hbm, u_ext_hbm),
             lambda: work(iidx_hbm, itabT_hbm, i_ext_hbm))


def _combine_body(u_ext_hbm, i_ext_hbm, w_hbm, b_hbm, out_hbm,
                  ubuf, ubuf1, ibuf, ibuf1, colbuf, out_v, wv, bv,
                  sem_u, sem_i):
    wid = lax.axis_index("s") * NUM_CORES + lax.axis_index("c")
    base = wid * B_PER_W

    pltpu.sync_copy(w_hbm, wv)
    pltpu.sync_copy(b_hbm, bv)

    iota = lax.iota(jnp.int32, LANES)
    iota16 = iota * LANES
    w_lo = wv[pl.ds(0, LANES)]
    w_hi = wv[pl.ds(LANES, LANES)]
    b_vec = bv[...]

    ubufs, ibufs = [ubuf, ubuf1], [ibuf, ibuf1]
    nchunks = B_PER_W // WIN

    def fire2(c, p):
        return [pltpu.async_copy(u_ext_hbm.at[pl.ds(base + c * WIN, WIN)],
                                 ubufs[p], sem_u),
                pltpu.async_copy(i_ext_hbm.at[pl.ds(base + c * WIN, WIN)],
                                 ibufs[p], sem_i)]

    cps = fire2(0, 0)
    for c in range(nchunks):
        if c + 1 < nchunks:
            nxt = fire2(c + 1, (c + 1) % 2)
        for cp in cps:
            cp.wait()
        ub, ib = ubufs[c % 2], ibufs[c % 2]

        def group(g, carry, c=c, ub=ub, ib=ib):
            j0 = g * LANES
            for e in range(LANES):
                j = j0 + e
                u_lo = ub[j, pl.ds(0, LANES)]
                u_hi = ub[j, pl.ds(LANES, LANES)]
                i_lo = ib[j, pl.ds(0, LANES)]
                i_hi = ib[j, pl.ds(LANES, LANES)]
                p = u_lo * i_lo * w_lo + u_hi * i_hi * w_hi
                plsc.store_scatter(colbuf, [iota16 + e], p)
            acc = colbuf[pl.ds(0, LANES)]
            for r in range(1, LANES):
                acc = acc + colbuf[pl.ds(r * LANES, LANES)]
            t = acc + b_vec
            sig = 1.0 / (1.0 + jnp.exp(-t))
            plsc.store_scatter(out_v, [c * WIN + g * LANES + iota], sig)
            return carry

        lax.fori_loop(0, WIN // LANES, group, 0)
        if c + 1 < nchunks:
            cps = nxt

    pltpu.sync_copy(out_v, out_hbm.at[pl.ds(base, B_PER_W)])


@jax.jit
def _gmf(user_indices, item_indices, utabT, itabT, w_flat, b_vec):
    mesh = plsc.VectorSubcoreMesh(core_axis_name="c", subcore_axis_name="s",
                                  num_cores=NUM_CORES, num_subcores=NUM_SUBCORES)
    stage = pl.kernel(
        _stage_body,
        out_type=[jax.ShapeDtypeStruct((EXT_ROWS, WIN), jnp.float32),
                  jax.ShapeDtypeStruct((EXT_ROWS, WIN), jnp.float32)],
        mesh=mesh,
        compiler_params=pltpu.CompilerParams(needs_layout_passes=False),
        scratch_types=[
            pltpu.VMEM((BATCH + LANES,), jnp.int32),
            pltpu.VMEM((HISTN,), jnp.int32),
            pltpu.VMEM((HISTN,), jnp.int32),
            pltpu.VMEM((BATCH + LANES,), jnp.int32),
            pltpu.VMEM((LATENT, WIN), jnp.float32),
            pltpu.VMEM((LATENT, WIN), jnp.float32),
            pltpu.VMEM((LATENT, WIN), jnp.float32),
            pltpu.VMEM((LATENT, WIN), jnp.float32),
            pltpu.VMEM((LATENT, WIN), jnp.float32),
            pltpu.VMEM((LATENT, WIN), jnp.float32),
            pltpu.VMEM((LATENT, WIN), jnp.float32),
            pltpu.VMEM((LATENT, WIN), jnp.float32),
            pltpu.VMEM((LATENT, WIN), jnp.float32),
            pltpu.VMEM((LATENT, WIN), jnp.float32),
            pltpu.VMEM((2 * WIN, WIN), jnp.float32),
            pltpu.VMEM((2, WIN), jnp.int32),
            pltpu.SemaphoreType.DMA,
            pltpu.SemaphoreType.DMA,
            pltpu.SemaphoreType.DMA,
            pltpu.SemaphoreType.DMA,
            pltpu.SemaphoreType.DMA,
            pltpu.SemaphoreType.DMA,
            pltpu.SemaphoreType.DMA,
            pltpu.SemaphoreType.DMA,
            pltpu.SemaphoreType.DMA,
            pltpu.SemaphoreType.DMA,
            pltpu.SemaphoreType.DMA,
            pltpu.SemaphoreType.DMA,
        ],
    )
    u_ext, i_ext = stage(user_indices, item_indices, utabT, itabT)
    combine = pl.kernel(
        _combine_body,
        out_type=jax.ShapeDtypeStruct((BATCH,), jnp.float32),
        mesh=mesh,
        compiler_params=pltpu.CompilerParams(needs_layout_passes=False),
        scratch_types=[
            pltpu.VMEM((WIN, WIN), jnp.float32),
            pltpu.VMEM((WIN, WIN), jnp.float32),
            pltpu.VMEM((WIN, WIN), jnp.float32),
            pltpu.VMEM((WIN, WIN), jnp.float32),
            pltpu.VMEM((LANES * LANES,), jnp.float32),
            pltpu.VMEM((B_PER_W,), jnp.float32),
            pltpu.VMEM((LATENT,), jnp.float32),
            pltpu.VMEM((LANES,), jnp.float32),
            pltpu.SemaphoreType.DMA,
            pltpu.SemaphoreType.DMA,
        ],
    )
    return combine(u_ext, i_ext, w_flat, b_vec)


def kernel(user_indices, item_indices, user_table, item_table, W, b):
    utabT = user_table.T
    itabT = item_table.T
    w_flat = W.reshape(LATENT)
    b_vec = jnp.broadcast_to(b, (LANES,))
    out = _gmf(user_indices.astype(jnp.int32), item_indices.astype(jnp.int32),
               utabT, itabT, w_flat, b_vec)
    return out.reshape(BATCH, 1)
